# baseline (speedup 1.0000x reference)
import functools
import math

import jax
import jax.numpy as jnp
from jax import lax
from jax.experimental import pallas as pl
from jax.experimental.pallas import tpu as pltpu
from jax.experimental.pallas import tpu_sc as plsc

B = 4096
L = 50
V = 100001
D = 32
NC = 2
NS = 16
NW = NC * NS
CB = 128
NCHUNK = B // CB
NSPLIT = 8
RCH = 12504
INV_L = 1.0 / L
RSQ = 1.0 / math.sqrt(1.0 + 1e-3)

SCRATCH = [
    pltpu.VMEM((V,), jnp.float32),
    pltpu.VMEM((L, CB), jnp.int32),
    pltpu.VMEM((L, CB), jnp.int32),
    pltpu.VMEM((B,), jnp.float32),
    pltpu.VMEM((B,), jnp.float32),
    pltpu.VMEM((B,), jnp.int32),
    pltpu.VMEM((48,), jnp.float32),
    pltpu.SemaphoreType.DMA,
    pltpu.SemaphoreType.DMA,
    pltpu.SemaphoreType.DMA,
]


def _body(idx_t, c0, c1, c2, c3, price, params,
          htab_t, t0, t1, t2, t3, out_t,
          row_v, idx0, idx1, colb, price_v, cidx_v, par_v,
          semr, sem0, sem1):
  cid = lax.axis_index("c")
  sid = lax.axis_index("s")
  d = sid * NC + cid

  def row_load(tab_t):
    pltpu.async_copy(tab_t.at[d], row_v, semr)

  def row_wait(tab_t):
    pltpu.make_async_copy(tab_t.at[d], row_v, semr).wait()

  row_load(htab_t)
  pltpu.sync_copy(price.at[:], price_v)
  pltpu.sync_copy(params.at[:], par_v)
  pltpu.async_copy(idx_t.at[0], idx0, sem0)

  par0 = par_v[pl.ds(0, 16)]
  gscale = par0[0] * RSQ
  bet = par0[1]
  dsplat = jnp.full((16,), 16 + d, jnp.int32)
  wsp = plsc.load_gather(par_v, [dsplat])

  def conti_step(k, _):
    bnv = price_v[pl.ds(16 * k, 16)] * gscale + bet
    colb[pl.ds(16 * k, 16)] = bnv * wsp
    return 0

  lax.fori_loop(0, B // 16, conti_step, 0)
  pltpu.sync_copy(colb, out_t.at[160 + d])

  row_wait(htab_t)

  def hist_block(buf, c):
    for g in range(CB // 16):
      iv = buf[0, pl.ds(g * 16, 16)]
      acc = plsc.load_gather(row_v, [iv])
      for j in range(1, L):
        iv = buf[j, pl.ds(g * 16, 16)]
        acc = acc + plsc.load_gather(row_v, [iv])
      colb[pl.ds(c * CB + g * 16, 16)] = acc * INV_L

  def hist_step(h, _):
    c_a = 2 * h
    c_b = 2 * h + 1
    pltpu.async_copy(idx_t.at[c_b], idx1, sem1)
    pltpu.make_async_copy(idx_t.at[c_a], idx0, sem0).wait()
    hist_block(idx0, c_a)
    pltpu.async_copy(idx_t.at[c_b + 1], idx0, sem0)
    pltpu.make_async_copy(idx_t.at[c_b], idx1, sem1).wait()
    hist_block(idx1, c_b)
    return 0

  lax.fori_loop(0, NCHUNK // 2 - 1, hist_step, 0)
  pltpu.async_copy(idx_t.at[NCHUNK - 1], idx1, sem1)
  pltpu.make_async_copy(idx_t.at[NCHUNK - 2], idx0, sem0).wait()
  hist_block(idx0, NCHUNK - 2)
  pltpu.make_async_copy(idx_t.at[NCHUNK - 1], idx1, sem1).wait()
  hist_block(idx1, NCHUNK - 1)
  pltpu.sync_copy(colb, out_t.at[d])

  def cate_phase(tab_t, cref, out_row):
    row_load(tab_t)
    pltpu.sync_copy(cref.at[:], cidx_v)
    row_wait(tab_t)

    def step(k, _):
      iv = cidx_v[pl.ds(16 * k, 16)]
      colb[pl.ds(16 * k, 16)] = plsc.load_gather(row_v, [iv])
      return 0

    lax.fori_loop(0, B // 16, step, 0)
    pltpu.sync_copy(colb, out_t.at[out_row])

  cate_phase(t0, c0, 32 + d)
  cate_phase(t1, c1, 64 + d)
  cate_phase(t2, c2, 96 + d)
  cate_phase(t3, c3, 128 + d)


@functools.partial(
    pl.kernel,
    out_type=jax.ShapeDtypeStruct((6 * D, B), jnp.float32),
    mesh=plsc.VectorSubcoreMesh(core_axis_name="c", subcore_axis_name="s",
                                num_cores=NC, num_subcores=NS),
    compiler_params=pltpu.CompilerParams(needs_layout_passes=False),
    scratch_types=SCRATCH,
)
def _sc_model(*refs):
  _body(*refs)


def kernel(hist_item, cate_0, cate_1, cate_2, cate_3, price,
           hist_table, cate_table_0, cate_table_1, cate_table_2, cate_table_3,
           bn_gamma, bn_beta, dense_W):
  params = jnp.zeros((48,), jnp.float32)
  params = params.at[0].set(bn_gamma[0]).at[1].set(bn_beta[0])
  params = params.at[16:48].set(dense_W.reshape(-1))
  idx3 = hist_item.T.reshape(L, NCHUNK, CB).transpose(1, 0, 2)
  out_t = _sc_model(
      idx3, cate_0.reshape(-1), cate_1.reshape(-1),
      cate_2.reshape(-1), cate_3.reshape(-1), price.reshape(-1), params,
      hist_table.T, cate_table_0.T, cate_table_1.T, cate_table_2.T,
      cate_table_3.T)
  return out_t.T
